# baseline (speedup 1.0000x reference)
import functools

import jax
import jax.numpy as jnp
from jax import lax
from jax.experimental import pallas as pl
from jax.experimental.pallas import tpu as pltpu
from jax.experimental.pallas import tpu_sc as plsc

N_NODES = 10000
N_EDGES = 320000
D = 128

NC = 2
NS = 16
NW = NC * NS
EPT = N_EDGES // NW
CHUNK = 125
NCH = EPT // CHUNK
GC = 16
NG = NCH // GC
WCH = 80
NWC = N_NODES // WCH
WPT = -(-NWC // NS)

_mesh = plsc.VectorSubcoreMesh(
    core_axis_name="c", subcore_axis_name="s", num_cores=NC, num_subcores=NS)


def _zero_1d(ref, n):
    def body(i, _):
        ref[pl.ds(i * 16, 16)] = jnp.zeros((16,), jnp.float32)
        return 0
    lax.fori_loop(0, n // 16, body, 0)


def _zero_2d(ref, rows, cols):
    nlc = cols // 16
    def body(i, _):
        ref[i // nlc, pl.ds((i % nlc) * 16, 16)] = jnp.zeros((16,), jnp.float32)
        return 0
    lax.fori_loop(0, rows * nlc, body, 0)


def _deg_body(src_hbm, dst_hbm, oo0, oi0, oo1, oi1,
              idx_s, idx_d, ones_v, zbuf, dego, degi):
    c = lax.axis_index("c")
    s = lax.axis_index("s")
    wid = c * NS + s
    pltpu.sync_copy(src_hbm.at[wid], idx_s)
    pltpu.sync_copy(dst_hbm.at[wid], idx_d)

    def fill_ones(i, _):
        ones_v[pl.ds(i * 16, 16)] = jnp.ones((16,), jnp.float32)
        return 0
    lax.fori_loop(0, CHUNK // 16, fill_ones, 0)
    if CHUNK % 16:
        ones_v[pl.ds(CHUNK - 16, 16)] = jnp.ones((16,), jnp.float32)

    @pl.when(s == 0)
    def _():
        _zero_1d(zbuf, N_NODES)
        pltpu.sync_copy(zbuf, dego)
        pltpu.sync_copy(zbuf, degi)

    plsc.subcore_barrier()

    def body(j, _):
        pltpu.sync_copy(ones_v, dego.at[idx_s.at[j]], add=True)
        pltpu.sync_copy(ones_v, degi.at[idx_d.at[j]], add=True)
        return 0
    lax.fori_loop(0, NCH, body, 0)

    plsc.subcore_barrier()

    @pl.when(s == 0)
    def _():
        pltpu.sync_copy(dego, zbuf)

        @pl.when(c == 0)
        def _():
            pltpu.sync_copy(zbuf, oo0)

        @pl.when(c == 1)
        def _():
            pltpu.sync_copy(zbuf, oo1)

    @pl.when(s == 1)
    def _():
        pltpu.sync_copy(degi, zbuf)

        @pl.when(c == 0)
        def _():
            pltpu.sync_copy(zbuf, oi0)

        @pl.when(c == 1)
        def _():
            pltpu.sync_copy(zbuf, oi1)


def _agg_body(xs_hbm, src_hbm, dst_hbm, out_hbm,
              idx_s, idx_d, rows_a, rows_b, wv, acc,
              sem_ga, sem_gb, sem_sa, sem_sb):
    c = lax.axis_index("c")
    s = lax.axis_index("s")
    wid = c * NS + s
    _zero_2d(wv, WCH, D)
    for t in range(WPT):
        i = s + NS * t

        @pl.when(i < NWC)
        def _():
            pltpu.sync_copy(wv, acc.at[pl.ds(i * WCH, WCH)])

    plsc.subcore_barrier()

    def group(g, _):
        pltpu.sync_copy(src_hbm.at[wid, g], idx_s)
        pltpu.sync_copy(dst_hbm.at[wid, g], idx_d)
        pltpu.async_copy(xs_hbm.at[idx_s.at[0]], rows_a, sem_ga)

        def pair(p, _):
            j = 2 * p
            pltpu.make_async_copy(xs_hbm.at[idx_s.at[j]], rows_a, sem_ga).wait()
            pltpu.async_copy(rows_a, acc.at[idx_d.at[j]], sem_sa, add=True)

            @pl.when(j > 0)
            def _():
                pltpu.make_async_copy(
                    rows_b, acc.at[idx_d.at[j - 1]], sem_sb).wait()
            pltpu.async_copy(xs_hbm.at[idx_s.at[j + 1]], rows_b, sem_gb)
            pltpu.make_async_copy(
                xs_hbm.at[idx_s.at[j + 1]], rows_b, sem_gb).wait()
            pltpu.async_copy(rows_b, acc.at[idx_d.at[j + 1]], sem_sb, add=True)
            pltpu.make_async_copy(rows_a, acc.at[idx_d.at[j]], sem_sa).wait()

            @pl.when(j + 2 < GC)
            def _():
                pltpu.async_copy(xs_hbm.at[idx_s.at[j + 2]], rows_a, sem_ga)
            return 0
        lax.fori_loop(0, GC // 2, pair, 0)
        pltpu.make_async_copy(rows_b, acc.at[idx_d.at[GC - 1]], sem_sb).wait()
        return 0
    lax.fori_loop(0, NG, group, 0)

    plsc.subcore_barrier()
    for t in range(WPT):
        i = s + NS * t

        @pl.when(i < NWC)
        def _():
            pltpu.sync_copy(acc.at[pl.ds(i * WCH, WCH)], wv)
            pltpu.sync_copy(wv, out_hbm.at[c, pl.ds(i * WCH, WCH)])


_deg_call = pl.kernel(
    _deg_body,
    out_type=tuple(jax.ShapeDtypeStruct((N_NODES,), jnp.float32)
                   for _ in range(4)),
    mesh=_mesh,
    scratch_types=[
        pltpu.VMEM((NCH, CHUNK), jnp.int32),
        pltpu.VMEM((NCH, CHUNK), jnp.int32),
        pltpu.VMEM((CHUNK,), jnp.float32),
        pltpu.VMEM((N_NODES,), jnp.float32),
        pltpu.VMEM_SHARED((N_NODES,), jnp.float32),
        pltpu.VMEM_SHARED((N_NODES,), jnp.float32),
    ],
)

_agg_call = pl.kernel(
    _agg_body,
    out_type=jax.ShapeDtypeStruct((NC, N_NODES, D), jnp.float32),
    mesh=_mesh,
    scratch_types=[
        pltpu.VMEM((GC, CHUNK), jnp.int32),
        pltpu.VMEM((GC, CHUNK), jnp.int32),
        pltpu.VMEM((CHUNK, D), jnp.float32),
        pltpu.VMEM((CHUNK, D), jnp.float32),
        pltpu.VMEM((WCH, D), jnp.float32),
        pltpu.VMEM_SHARED((N_NODES, D), jnp.float32),
        pltpu.SemaphoreType.DMA,
        pltpu.SemaphoreType.DMA,
        pltpu.SemaphoreType.DMA,
        pltpu.SemaphoreType.DMA,
    ],
)


BLK = 1000
NBLK = N_NODES // BLK


def _norm(deg):
    return jnp.where(deg > 0.0, lax.rsqrt(jnp.maximum(deg, 1.0)), 0.0)


def _prescale_body(h_ref, do0_ref, do1_ref, out_ref):
    dego = do0_ref[...] + do1_ref[...]
    ns = _norm(dego[0, 0])
    out_ref[...] = h_ref[...] * ns[:, None]


def _post_body(part_ref, do0_ref, do1_ref, di0_ref, di1_ref, w_ref, b_ref,
               out_ref, *, apply_src):
    p = part_ref[...]
    agg = p[0] + p[1]
    degi = di0_ref[...] + di1_ref[...]
    nd = _norm(degi[0, 0])[:, None]
    y = jnp.dot(agg * nd, w_ref[...],
                preferred_element_type=jnp.float32,
                precision=lax.Precision.HIGHEST) + b_ref[...]
    y = jnp.maximum(y, 0.0)
    if apply_src:
        dego = do0_ref[...] + do1_ref[...]
        y = y * _norm(dego[0, 0])[:, None]
    out_ref[...] = y


_deg_spec = pl.BlockSpec((1, 1, BLK), lambda i: (i, 0, 0))


def _prescale(h, do0, do1):
    return pl.pallas_call(
        _prescale_body,
        grid=(NBLK,),
        in_specs=[
            pl.BlockSpec((BLK, D), lambda i: (i, 0)),
            _deg_spec,
            _deg_spec,
        ],
        out_specs=pl.BlockSpec((BLK, D), lambda i: (i, 0)),
        out_shape=jax.ShapeDtypeStruct((N_NODES, D), jnp.float32),
    )(h, do0, do1)


def _post(part, do0, do1, di0, di1, w, b2d, apply_src):
    return pl.pallas_call(
        functools.partial(_post_body, apply_src=apply_src),
        grid=(NBLK,),
        in_specs=[
            pl.BlockSpec((NC, BLK, D), lambda i: (0, i, 0)),
            _deg_spec,
            _deg_spec,
            _deg_spec,
            _deg_spec,
            pl.BlockSpec((D, D), lambda i: (0, 0)),
            pl.BlockSpec((1, D), lambda i: (0, 0)),
        ],
        out_specs=pl.BlockSpec((BLK, D), lambda i: (i, 0)),
        out_shape=jax.ShapeDtypeStruct((N_NODES, D), jnp.float32),
    )(part, do0, do1, di0, di1, w, b2d)


def kernel(h, edge_index, W1, b1, W2, b2):
    ei = edge_index.astype(jnp.int32)
    src3 = ei[0].reshape(NW, NCH, CHUNK)
    dst3 = ei[1].reshape(NW, NCH, CHUNK)
    src4 = ei[0].reshape(NW, NG, GC, CHUNK)
    dst4 = ei[1].reshape(NW, NG, GC, CHUNK)
    dego0, degi0, dego1, degi1 = _deg_call(src3, dst3)
    r = lambda a: a.reshape(NBLK, 1, BLK)
    do0, do1, di0, di1 = r(dego0), r(dego1), r(degi0), r(degi1)
    xs1 = _prescale(h, do0, do1)
    part1 = _agg_call(xs1, src4, dst4)
    xs2 = _post(part1, do0, do1, di0, di1, W1, b1.reshape(1, D), True)
    part2 = _agg_call(xs2, src4, dst4)
    out = _post(part2, do0, do1, di0, di1, W2, b2.reshape(1, D), False)
    return out


# speedup vs baseline: 1.0494x; 1.0494x over previous
import functools

import jax
import jax.numpy as jnp
from jax import lax
from jax.experimental import pallas as pl
from jax.experimental.pallas import tpu as pltpu
from jax.experimental.pallas import tpu_sc as plsc

N_NODES = 10000
N_EDGES = 320000
D = 128

NC = 2
NS = 16
NW = NC * NS
EPT = N_EDGES // NW
CHUNK = 125
NCH = EPT // CHUNK
GC = 16
NG = NCH // GC
WCH = 80
NWC = N_NODES // WCH
WPT = -(-NWC // NS)

_mesh = plsc.VectorSubcoreMesh(
    core_axis_name="c", subcore_axis_name="s", num_cores=NC, num_subcores=NS)


def _zero_1d(ref, n):
    def body(i, _):
        ref[pl.ds(i * 16, 16)] = jnp.zeros((16,), jnp.float32)
        return 0
    lax.fori_loop(0, n // 16, body, 0)


def _zero_2d(ref, rows, cols):
    nlc = cols // 16
    def body(i, _):
        ref[i // nlc, pl.ds((i % nlc) * 16, 16)] = jnp.zeros((16,), jnp.float32)
        return 0
    lax.fori_loop(0, rows * nlc, body, 0)


ZCH = 624
ZTAIL = N_NODES - NS * ZCH


def _deg_body(src_hbm, dst_hbm, oo0, oi0, oo1, oi1,
              idx_s, idx_d, ones_v, zbuf, dego, degi, sem_a, sem_b, sem_z):
    c = lax.axis_index("c")
    s = lax.axis_index("s")
    wid = c * NS + s

    def fill_ones(i, _):
        ones_v[pl.ds(i * 16, 16)] = jnp.ones((16,), jnp.float32)
        return 0
    lax.fori_loop(0, CHUNK // 16, fill_ones, 0)
    if CHUNK % 16:
        ones_v[pl.ds(CHUNK - 16, 16)] = jnp.ones((16,), jnp.float32)
    _zero_1d(zbuf, ZCH)

    pltpu.async_copy(zbuf, dego.at[pl.ds(s * ZCH, ZCH)], sem_z)
    pltpu.async_copy(zbuf, degi.at[pl.ds(s * ZCH, ZCH)], sem_z)

    @pl.when(s == 0)
    def _():
        pltpu.async_copy(
            zbuf.at[pl.ds(0, ZTAIL)], dego.at[pl.ds(NS * ZCH, ZTAIL)], sem_z)
        pltpu.async_copy(
            zbuf.at[pl.ds(0, ZTAIL)], degi.at[pl.ds(NS * ZCH, ZTAIL)], sem_z)
        pltpu.make_async_copy(
            zbuf.at[pl.ds(0, ZTAIL)], dego.at[pl.ds(NS * ZCH, ZTAIL)],
            sem_z).wait()
        pltpu.make_async_copy(
            zbuf.at[pl.ds(0, ZTAIL)], degi.at[pl.ds(NS * ZCH, ZTAIL)],
            sem_z).wait()

    pltpu.make_async_copy(zbuf, dego.at[pl.ds(s * ZCH, ZCH)], sem_z).wait()
    pltpu.make_async_copy(zbuf, degi.at[pl.ds(s * ZCH, ZCH)], sem_z).wait()

    plsc.subcore_barrier()

    LAG = 4
    for g in range(NG):
        pltpu.sync_copy(src_hbm.at[wid, g], idx_s)
        pltpu.sync_copy(dst_hbm.at[wid, g], idx_d)

        def fire(j, _):
            pltpu.async_copy(ones_v, dego.at[idx_s.at[j]], sem_a, add=True)
            pltpu.async_copy(ones_v, degi.at[idx_d.at[j]], sem_b, add=True)

            @pl.when(j >= LAG)
            def _():
                pltpu.make_async_copy(
                    ones_v, dego.at[idx_s.at[0]], sem_a).wait()
                pltpu.make_async_copy(
                    ones_v, degi.at[idx_d.at[0]], sem_b).wait()
            return 0
        lax.fori_loop(0, GC, fire, 0)

        def drain(j, _):
            pltpu.make_async_copy(
                ones_v, dego.at[idx_s.at[0]], sem_a).wait()
            pltpu.make_async_copy(
                ones_v, degi.at[idx_d.at[0]], sem_b).wait()
            return 0
        lax.fori_loop(0, LAG, drain, 0)

    plsc.subcore_barrier()

    def _writeout(deg_ref, out_ref):
        pltpu.sync_copy(deg_ref.at[pl.ds(s * ZCH, ZCH)], zbuf)
        pltpu.sync_copy(zbuf, out_ref.at[pl.ds(s * ZCH, ZCH)])

        @pl.when(s == 0)
        def _():
            pltpu.sync_copy(deg_ref.at[pl.ds(NS * ZCH, ZTAIL)],
                            zbuf.at[pl.ds(0, ZTAIL)])
            pltpu.sync_copy(zbuf.at[pl.ds(0, ZTAIL)],
                            out_ref.at[pl.ds(NS * ZCH, ZTAIL)])

    @pl.when(c == 0)
    def _():
        _writeout(dego, oo0)
        _writeout(degi, oi0)

    @pl.when(c == 1)
    def _():
        _writeout(dego, oo1)
        _writeout(degi, oi1)


def _agg_body(xs_hbm, zeros_hbm, src_hbm, dst_hbm, out_hbm,
              idx_s0, idx_d0, idx_s1, idx_d1, rows_a, rows_b, acc,
              sem_ga, sem_gb, sem_sa, sem_sb, sem_p, sem_z):
    c = lax.axis_index("c")
    s = lax.axis_index("s")
    wid = c * NS + s
    za = rows_a.at[pl.ds(0, WCH)]
    pltpu.sync_copy(zeros_hbm, za)
    for t in range(WPT):
        i = s + NS * t

        @pl.when(i < NWC)
        def _():
            pltpu.async_copy(za, acc.at[pl.ds(i * WCH, WCH)], sem_z)
    for t in range(WPT):
        i = s + NS * t

        @pl.when(i < NWC)
        def _():
            pltpu.make_async_copy(
                za, acc.at[pl.ds(i * WCH, WCH)], sem_z).wait()

    plsc.subcore_barrier()

    pltpu.sync_copy(src_hbm.at[wid, 0], idx_s0)
    pltpu.sync_copy(dst_hbm.at[wid, 0], idx_d0)
    for g in range(NG):
        cur_s, cur_d = (idx_s0, idx_d0) if g % 2 == 0 else (idx_s1, idx_d1)
        nxt_s, nxt_d = (idx_s1, idx_d1) if g % 2 == 0 else (idx_s0, idx_d0)
        if g + 1 < NG:
            pltpu.async_copy(src_hbm.at[wid, g + 1], nxt_s, sem_p)
            pltpu.async_copy(dst_hbm.at[wid, g + 1], nxt_d, sem_p)
        pltpu.async_copy(xs_hbm.at[cur_s.at[0]], rows_a, sem_ga)

        def pair(p, _, cur_s=cur_s, cur_d=cur_d):
            j = 2 * p
            pltpu.make_async_copy(xs_hbm.at[cur_s.at[j]], rows_a, sem_ga).wait()
            pltpu.async_copy(rows_a, acc.at[cur_d.at[j]], sem_sa, add=True)

            @pl.when(j > 0)
            def _():
                pltpu.make_async_copy(
                    rows_b, acc.at[cur_d.at[j - 1]], sem_sb).wait()
            pltpu.async_copy(xs_hbm.at[cur_s.at[j + 1]], rows_b, sem_gb)
            pltpu.make_async_copy(
                xs_hbm.at[cur_s.at[j + 1]], rows_b, sem_gb).wait()
            pltpu.async_copy(rows_b, acc.at[cur_d.at[j + 1]], sem_sb, add=True)
            pltpu.make_async_copy(rows_a, acc.at[cur_d.at[j]], sem_sa).wait()

            @pl.when(j + 2 < GC)
            def _():
                pltpu.async_copy(xs_hbm.at[cur_s.at[j + 2]], rows_a, sem_ga)
            return 0
        lax.fori_loop(0, GC // 2, pair, 0)
        pltpu.make_async_copy(rows_b, acc.at[cur_d.at[GC - 1]], sem_sb).wait()
        if g + 1 < NG:
            pltpu.make_async_copy(
                src_hbm.at[wid, g + 1], nxt_s, sem_p).wait()
            pltpu.make_async_copy(
                dst_hbm.at[wid, g + 1], nxt_d, sem_p).wait()

    plsc.subcore_barrier()
    bufs = (rows_a.at[pl.ds(0, WCH)], rows_b.at[pl.ds(0, WCH)])
    isems = (sem_ga, sem_gb)
    osems = (sem_sa, sem_sb)

    def _wo(t, fn):
        i = s + NS * t

        @pl.when(i < NWC)
        def _():
            fn(i, bufs[t % 2])

    def _in(t):
        _wo(t, lambda i, b: pltpu.async_copy(
            acc.at[pl.ds(i * WCH, WCH)], b, isems[t % 2]))

    def _wait_in(t):
        _wo(t, lambda i, b: pltpu.make_async_copy(
            acc.at[pl.ds(i * WCH, WCH)], b, isems[t % 2]).wait())

    def _out(t):
        _wo(t, lambda i, b: pltpu.async_copy(
            b, out_hbm.at[c, pl.ds(i * WCH, WCH)], osems[t % 2]))

    def _wait_out(t):
        _wo(t, lambda i, b: pltpu.make_async_copy(
            b, out_hbm.at[c, pl.ds(i * WCH, WCH)], osems[t % 2]).wait())

    _in(0)
    for t in range(WPT):
        _wait_in(t)
        _out(t)
        if t + 1 < WPT:
            if t >= 1:
                _wait_out(t - 1)
            _in(t + 1)
    _wait_out(WPT - 2)
    _wait_out(WPT - 1)


_deg_call = pl.kernel(
    _deg_body,
    out_type=tuple(jax.ShapeDtypeStruct((N_NODES,), jnp.float32)
                   for _ in range(4)),
    mesh=_mesh,
    scratch_types=[
        pltpu.VMEM((GC, CHUNK), jnp.int32),
        pltpu.VMEM((GC, CHUNK), jnp.int32),
        pltpu.VMEM((CHUNK,), jnp.float32),
        pltpu.VMEM((ZCH,), jnp.float32),
        pltpu.VMEM_SHARED((N_NODES,), jnp.float32),
        pltpu.VMEM_SHARED((N_NODES,), jnp.float32),
        pltpu.SemaphoreType.DMA,
        pltpu.SemaphoreType.DMA,
        pltpu.SemaphoreType.DMA,
    ],
)

_agg_call = pl.kernel(
    _agg_body,
    out_type=jax.ShapeDtypeStruct((NC, N_NODES, D), jnp.float32),
    mesh=_mesh,
    scratch_types=[
        pltpu.VMEM((GC, CHUNK), jnp.int32),
        pltpu.VMEM((GC, CHUNK), jnp.int32),
        pltpu.VMEM((GC, CHUNK), jnp.int32),
        pltpu.VMEM((GC, CHUNK), jnp.int32),
        pltpu.VMEM((CHUNK, D), jnp.float32),
        pltpu.VMEM((CHUNK, D), jnp.float32),
        pltpu.VMEM_SHARED((N_NODES, D), jnp.float32),
        pltpu.SemaphoreType.DMA,
        pltpu.SemaphoreType.DMA,
        pltpu.SemaphoreType.DMA,
        pltpu.SemaphoreType.DMA,
        pltpu.SemaphoreType.DMA,
        pltpu.SemaphoreType.DMA,
    ],
)


BLK = 1000
NBLK = N_NODES // BLK


def _norm(deg):
    return jnp.where(deg > 0.0, lax.rsqrt(jnp.maximum(deg, 1.0)), 0.0)


def _prescale_body(h_ref, do0_ref, do1_ref, out_ref):
    dego = do0_ref[...] + do1_ref[...]
    ns = _norm(dego[0, 0])
    out_ref[...] = h_ref[...] * ns[:, None]


def _post_body(part_ref, do0_ref, do1_ref, di0_ref, di1_ref, w_ref, b_ref,
               out_ref, *, apply_src):
    p = part_ref[...]
    agg = p[0] + p[1]
    degi = di0_ref[...] + di1_ref[...]
    nd = _norm(degi[0, 0])[:, None]
    y = jnp.dot(agg * nd, w_ref[...],
                preferred_element_type=jnp.float32,
                precision=lax.Precision.HIGHEST) + b_ref[...]
    y = jnp.maximum(y, 0.0)
    if apply_src:
        dego = do0_ref[...] + do1_ref[...]
        y = y * _norm(dego[0, 0])[:, None]
    out_ref[...] = y


_deg_spec = pl.BlockSpec((1, 1, BLK), lambda i: (i, 0, 0))


def _prescale(h, do0, do1):
    return pl.pallas_call(
        _prescale_body,
        grid=(NBLK,),
        in_specs=[
            pl.BlockSpec((BLK, D), lambda i: (i, 0)),
            _deg_spec,
            _deg_spec,
        ],
        out_specs=pl.BlockSpec((BLK, D), lambda i: (i, 0)),
        out_shape=jax.ShapeDtypeStruct((N_NODES, D), jnp.float32),
    )(h, do0, do1)


def _post(part, do0, do1, di0, di1, w, b2d, apply_src):
    return pl.pallas_call(
        functools.partial(_post_body, apply_src=apply_src),
        grid=(NBLK,),
        in_specs=[
            pl.BlockSpec((NC, BLK, D), lambda i: (0, i, 0)),
            _deg_spec,
            _deg_spec,
            _deg_spec,
            _deg_spec,
            pl.BlockSpec((D, D), lambda i: (0, 0)),
            pl.BlockSpec((1, D), lambda i: (0, 0)),
        ],
        out_specs=pl.BlockSpec((BLK, D), lambda i: (i, 0)),
        out_shape=jax.ShapeDtypeStruct((N_NODES, D), jnp.float32),
    )(part, do0, do1, di0, di1, w, b2d)


def kernel(h, edge_index, W1, b1, W2, b2):
    ei = edge_index.astype(jnp.int32)
    src4 = ei[0].reshape(NW, NG, GC, CHUNK)
    dst4 = ei[1].reshape(NW, NG, GC, CHUNK)
    zeros2d = jnp.zeros((WCH, D), jnp.float32)
    dego0, degi0, dego1, degi1 = _deg_call(src4, dst4)
    r = lambda a: a.reshape(NBLK, 1, BLK)
    do0, do1, di0, di1 = r(dego0), r(dego1), r(degi0), r(degi1)
    xs1 = _prescale(h, do0, do1)
    part1 = _agg_call(xs1, zeros2d, src4, dst4)
    xs2 = _post(part1, do0, do1, di0, di1, W1, b1.reshape(1, D), True)
    part2 = _agg_call(xs2, zeros2d, src4, dst4)
    out = _post(part2, do0, do1, di0, di1, W2, b2.reshape(1, D), False)
    return out


# speedup vs baseline: 1.0657x; 1.0156x over previous
import functools

import jax
import jax.numpy as jnp
from jax import lax
from jax.experimental import pallas as pl
from jax.experimental.pallas import tpu as pltpu
from jax.experimental.pallas import tpu_sc as plsc

N_NODES = 10000
N_EDGES = 320000
D = 128

NC = 2
NS = 16
NW = NC * NS
EPT = N_EDGES // NW
CHUNK = 125
NCH = EPT // CHUNK
GC = 16
NG = NCH // GC
WCH = 80
NWC = N_NODES // WCH
WPT = -(-NWC // NS)

_mesh = plsc.VectorSubcoreMesh(
    core_axis_name="c", subcore_axis_name="s", num_cores=NC, num_subcores=NS)


def _zero_1d(ref, n):
    def body(i, _):
        ref[pl.ds(i * 16, 16)] = jnp.zeros((16,), jnp.float32)
        return 0
    lax.fori_loop(0, n // 16, body, 0)


def _zero_2d(ref, rows, cols):
    nlc = cols // 16
    def body(i, _):
        ref[i // nlc, pl.ds((i % nlc) * 16, 16)] = jnp.zeros((16,), jnp.float32)
        return 0
    lax.fori_loop(0, rows * nlc, body, 0)


ZCH = 624
ZTAIL = N_NODES - NS * ZCH


def _deg_body(src_hbm, dst_hbm, oo0, oi0, oo1, oi1,
              idx_s, idx_d, ones_v, zbuf, dego, degi, sem_a, sem_b, sem_z):
    c = lax.axis_index("c")
    s = lax.axis_index("s")
    wid = c * NS + s

    def fill_ones(i, _):
        ones_v[pl.ds(i * 16, 16)] = jnp.ones((16,), jnp.float32)
        return 0
    lax.fori_loop(0, CHUNK // 16, fill_ones, 0)
    if CHUNK % 16:
        ones_v[pl.ds(CHUNK - 16, 16)] = jnp.ones((16,), jnp.float32)
    _zero_1d(zbuf, ZCH)

    pltpu.async_copy(zbuf, dego.at[pl.ds(s * ZCH, ZCH)], sem_z)
    pltpu.async_copy(zbuf, degi.at[pl.ds(s * ZCH, ZCH)], sem_z)

    @pl.when(s == 0)
    def _():
        pltpu.async_copy(
            zbuf.at[pl.ds(0, ZTAIL)], dego.at[pl.ds(NS * ZCH, ZTAIL)], sem_z)
        pltpu.async_copy(
            zbuf.at[pl.ds(0, ZTAIL)], degi.at[pl.ds(NS * ZCH, ZTAIL)], sem_z)
        pltpu.make_async_copy(
            zbuf.at[pl.ds(0, ZTAIL)], dego.at[pl.ds(NS * ZCH, ZTAIL)],
            sem_z).wait()
        pltpu.make_async_copy(
            zbuf.at[pl.ds(0, ZTAIL)], degi.at[pl.ds(NS * ZCH, ZTAIL)],
            sem_z).wait()

    pltpu.make_async_copy(zbuf, dego.at[pl.ds(s * ZCH, ZCH)], sem_z).wait()
    pltpu.make_async_copy(zbuf, degi.at[pl.ds(s * ZCH, ZCH)], sem_z).wait()

    plsc.subcore_barrier()

    LAG = 4
    for g in range(NG):
        pltpu.sync_copy(src_hbm.at[wid, g], idx_s)
        pltpu.sync_copy(dst_hbm.at[wid, g], idx_d)

        def fire(j, _):
            pltpu.async_copy(ones_v, dego.at[idx_s.at[j]], sem_a, add=True)
            pltpu.async_copy(ones_v, degi.at[idx_d.at[j]], sem_b, add=True)

            @pl.when(j >= LAG)
            def _():
                pltpu.make_async_copy(
                    ones_v, dego.at[idx_s.at[0]], sem_a).wait()
                pltpu.make_async_copy(
                    ones_v, degi.at[idx_d.at[0]], sem_b).wait()
            return 0
        lax.fori_loop(0, GC, fire, 0)

        def drain(j, _):
            pltpu.make_async_copy(
                ones_v, dego.at[idx_s.at[0]], sem_a).wait()
            pltpu.make_async_copy(
                ones_v, degi.at[idx_d.at[0]], sem_b).wait()
            return 0
        lax.fori_loop(0, LAG, drain, 0)

    plsc.subcore_barrier()

    def _writeout(deg_ref, out_ref):
        pltpu.sync_copy(deg_ref.at[pl.ds(s * ZCH, ZCH)], zbuf)
        pltpu.sync_copy(zbuf, out_ref.at[pl.ds(s * ZCH, ZCH)])

        @pl.when(s == 0)
        def _():
            pltpu.sync_copy(deg_ref.at[pl.ds(NS * ZCH, ZTAIL)],
                            zbuf.at[pl.ds(0, ZTAIL)])
            pltpu.sync_copy(zbuf.at[pl.ds(0, ZTAIL)],
                            out_ref.at[pl.ds(NS * ZCH, ZTAIL)])

    @pl.when(c == 0)
    def _():
        _writeout(dego, oo0)
        _writeout(degi, oi0)

    @pl.when(c == 1)
    def _():
        _writeout(dego, oo1)
        _writeout(degi, oi1)


def _agg_body(xs_hbm, zeros_hbm, src_hbm, dst_hbm, out_hbm,
              idx_s0, idx_d0, idx_s1, idx_d1, rows_a, rows_b, acc,
              sem_ga, sem_gb, sem_sa, sem_sb, sem_p, sem_z):
    c = lax.axis_index("c")
    s = lax.axis_index("s")
    wid = c * NS + s
    za = rows_a.at[pl.ds(0, WCH)]
    pltpu.sync_copy(zeros_hbm, za)
    for t in range(WPT):
        i = s + NS * t

        @pl.when(i < NWC)
        def _():
            pltpu.async_copy(za, acc.at[pl.ds(i * WCH, WCH)], sem_z)
    for t in range(WPT):
        i = s + NS * t

        @pl.when(i < NWC)
        def _():
            pltpu.make_async_copy(
                za, acc.at[pl.ds(i * WCH, WCH)], sem_z).wait()

    plsc.subcore_barrier()

    pltpu.sync_copy(src_hbm.at[wid, 0], idx_s0)
    pltpu.sync_copy(dst_hbm.at[wid, 0], idx_d0)
    pltpu.async_copy(xs_hbm.at[idx_s0.at[0]], rows_a, sem_ga)
    for g in range(NG):
        cur_s, cur_d = (idx_s0, idx_d0) if g % 2 == 0 else (idx_s1, idx_d1)
        nxt_s, nxt_d = (idx_s1, idx_d1) if g % 2 == 0 else (idx_s0, idx_d0)
        if g + 1 < NG:
            pltpu.async_copy(src_hbm.at[wid, g + 1], nxt_s, sem_p)
            pltpu.async_copy(dst_hbm.at[wid, g + 1], nxt_d, sem_p)

        def pair(p, _, cur_s=cur_s, cur_d=cur_d, first=(g == 0)):
            j = 2 * p

            def wait_sb():
                pltpu.make_async_copy(
                    rows_b, acc.at[cur_d.at[0]], sem_sb).wait()
            pltpu.make_async_copy(xs_hbm.at[cur_s.at[j]], rows_a, sem_ga).wait()
            pltpu.async_copy(rows_a, acc.at[cur_d.at[j]], sem_sa, add=True)
            if first:
                @pl.when(j > 0)
                def _():
                    wait_sb()
            else:
                wait_sb()
            pltpu.async_copy(xs_hbm.at[cur_s.at[j + 1]], rows_b, sem_gb)
            pltpu.make_async_copy(
                xs_hbm.at[cur_s.at[j + 1]], rows_b, sem_gb).wait()
            pltpu.async_copy(rows_b, acc.at[cur_d.at[j + 1]], sem_sb, add=True)
            pltpu.make_async_copy(rows_a, acc.at[cur_d.at[0]], sem_sa).wait()

            @pl.when(j + 2 < GC)
            def _():
                pltpu.async_copy(xs_hbm.at[cur_s.at[j + 2]], rows_a, sem_ga)
            return 0
        lax.fori_loop(0, GC // 2, pair, 0)
        if g + 1 < NG:
            pltpu.make_async_copy(
                src_hbm.at[wid, g + 1], nxt_s, sem_p).wait()
            pltpu.make_async_copy(
                dst_hbm.at[wid, g + 1], nxt_d, sem_p).wait()
            pltpu.async_copy(xs_hbm.at[nxt_s.at[0]], rows_a, sem_ga)
    pltpu.make_async_copy(rows_b, acc.at[idx_d0.at[0]], sem_sb).wait()

    plsc.subcore_barrier()
    bufs = (rows_a.at[pl.ds(0, WCH)], rows_b.at[pl.ds(0, WCH)])
    isems = (sem_ga, sem_gb)
    osems = (sem_sa, sem_sb)

    def _wo(t, fn):
        i = s + NS * t

        @pl.when(i < NWC)
        def _():
            fn(i, bufs[t % 2])

    def _in(t):
        _wo(t, lambda i, b: pltpu.async_copy(
            acc.at[pl.ds(i * WCH, WCH)], b, isems[t % 2]))

    def _wait_in(t):
        _wo(t, lambda i, b: pltpu.make_async_copy(
            acc.at[pl.ds(i * WCH, WCH)], b, isems[t % 2]).wait())

    def _out(t):
        _wo(t, lambda i, b: pltpu.async_copy(
            b, out_hbm.at[c, pl.ds(i * WCH, WCH)], osems[t % 2]))

    def _wait_out(t):
        _wo(t, lambda i, b: pltpu.make_async_copy(
            b, out_hbm.at[c, pl.ds(i * WCH, WCH)], osems[t % 2]).wait())

    _in(0)
    for t in range(WPT):
        _wait_in(t)
        _out(t)
        if t + 1 < WPT:
            if t >= 1:
                _wait_out(t - 1)
            _in(t + 1)
    _wait_out(WPT - 2)
    _wait_out(WPT - 1)


_deg_call = pl.kernel(
    _deg_body,
    out_type=tuple(jax.ShapeDtypeStruct((N_NODES,), jnp.float32)
                   for _ in range(4)),
    mesh=_mesh,
    scratch_types=[
        pltpu.VMEM((GC, CHUNK), jnp.int32),
        pltpu.VMEM((GC, CHUNK), jnp.int32),
        pltpu.VMEM((CHUNK,), jnp.float32),
        pltpu.VMEM((ZCH,), jnp.float32),
        pltpu.VMEM_SHARED((N_NODES,), jnp.float32),
        pltpu.VMEM_SHARED((N_NODES,), jnp.float32),
        pltpu.SemaphoreType.DMA,
        pltpu.SemaphoreType.DMA,
        pltpu.SemaphoreType.DMA,
    ],
)

_agg_call = pl.kernel(
    _agg_body,
    out_type=jax.ShapeDtypeStruct((NC, N_NODES, D), jnp.float32),
    mesh=_mesh,
    scratch_types=[
        pltpu.VMEM((GC, CHUNK), jnp.int32),
        pltpu.VMEM((GC, CHUNK), jnp.int32),
        pltpu.VMEM((GC, CHUNK), jnp.int32),
        pltpu.VMEM((GC, CHUNK), jnp.int32),
        pltpu.VMEM((CHUNK, D), jnp.float32),
        pltpu.VMEM((CHUNK, D), jnp.float32),
        pltpu.VMEM_SHARED((N_NODES, D), jnp.float32),
        pltpu.SemaphoreType.DMA,
        pltpu.SemaphoreType.DMA,
        pltpu.SemaphoreType.DMA,
        pltpu.SemaphoreType.DMA,
        pltpu.SemaphoreType.DMA,
        pltpu.SemaphoreType.DMA,
    ],
)


BLK = 1000
NBLK = N_NODES // BLK


def _norm(deg):
    return jnp.where(deg > 0.0, lax.rsqrt(jnp.maximum(deg, 1.0)), 0.0)


def _prescale_body(h_ref, do0_ref, do1_ref, out_ref):
    dego = do0_ref[...] + do1_ref[...]
    ns = _norm(dego[0, 0])
    out_ref[...] = h_ref[...] * ns[:, None]


def _post_body(part_ref, do0_ref, do1_ref, di0_ref, di1_ref, w_ref, b_ref,
               out_ref, *, apply_src):
    p = part_ref[...]
    agg = p[0] + p[1]
    degi = di0_ref[...] + di1_ref[...]
    nd = _norm(degi[0, 0])[:, None]
    y = jnp.dot(agg * nd, w_ref[...],
                preferred_element_type=jnp.float32,
                precision=lax.Precision.HIGHEST) + b_ref[...]
    y = jnp.maximum(y, 0.0)
    if apply_src:
        dego = do0_ref[...] + do1_ref[...]
        y = y * _norm(dego[0, 0])[:, None]
    out_ref[...] = y


_deg_spec = pl.BlockSpec((1, 1, BLK), lambda i: (i, 0, 0))


def _prescale(h, do0, do1):
    return pl.pallas_call(
        _prescale_body,
        grid=(NBLK,),
        in_specs=[
            pl.BlockSpec((BLK, D), lambda i: (i, 0)),
            _deg_spec,
            _deg_spec,
        ],
        out_specs=pl.BlockSpec((BLK, D), lambda i: (i, 0)),
        out_shape=jax.ShapeDtypeStruct((N_NODES, D), jnp.float32),
    )(h, do0, do1)


def _post(part, do0, do1, di0, di1, w, b2d, apply_src):
    return pl.pallas_call(
        functools.partial(_post_body, apply_src=apply_src),
        grid=(NBLK,),
        in_specs=[
            pl.BlockSpec((NC, BLK, D), lambda i: (0, i, 0)),
            _deg_spec,
            _deg_spec,
            _deg_spec,
            _deg_spec,
            pl.BlockSpec((D, D), lambda i: (0, 0)),
            pl.BlockSpec((1, D), lambda i: (0, 0)),
        ],
        out_specs=pl.BlockSpec((BLK, D), lambda i: (i, 0)),
        out_shape=jax.ShapeDtypeStruct((N_NODES, D), jnp.float32),
    )(part, do0, do1, di0, di1, w, b2d)


def kernel(h, edge_index, W1, b1, W2, b2):
    ei = edge_index.astype(jnp.int32)
    src4 = ei[0].reshape(NW, NG, GC, CHUNK)
    dst4 = ei[1].reshape(NW, NG, GC, CHUNK)
    zeros2d = jnp.zeros((WCH, D), jnp.float32)
    dego0, degi0, dego1, degi1 = _deg_call(src4, dst4)
    r = lambda a: a.reshape(NBLK, 1, BLK)
    do0, do1, di0, di1 = r(dego0), r(dego1), r(degi0), r(degi1)
    xs1 = _prescale(h, do0, do1)
    part1 = _agg_call(xs1, zeros2d, src4, dst4)
    xs2 = _post(part1, do0, do1, di0, di1, W1, b1.reshape(1, D), True)
    part2 = _agg_call(xs2, zeros2d, src4, dst4)
    out = _post(part2, do0, do1, di0, di1, W2, b2.reshape(1, D), False)
    return out
